# Initial kernel scaffold; baseline (speedup 1.0000x reference)
import jax
import jax.numpy as jnp
from jax.experimental import pallas as pl


def kernel(input_ids, emb, W_enc, b_enc, W_dec, b_dec, W_out, b_out):
    raise NotImplementedError("write your pallas kernel here")


# speedup vs baseline: 67.7455x; 67.7455x over previous
import functools

import jax
import jax.numpy as jnp
from jax import lax
from jax.experimental import pallas as pl
from jax.experimental.pallas import tpu as pltpu
from jax.experimental.pallas import tpu_sc as plsc

L = 16
DECAY = 0.9
TCHUNK = 256


def _sc_gather(emb, ids):
    T = ids.shape[0]
    V, E = emb.shape
    info = plsc.get_sparse_core_info()
    NW = info.num_cores * info.num_subcores
    b_per_w = T // NW
    mesh = plsc.VectorSubcoreMesh(core_axis_name="c", subcore_axis_name="s")

    @functools.partial(
        pl.kernel,
        mesh=mesh,
        out_type=jax.ShapeDtypeStruct((T, E), jnp.float32),
        scratch_types=[
            pltpu.VMEM((b_per_w,), jnp.int32),
            pltpu.VMEM((b_per_w, E), jnp.float32),
            pltpu.SemaphoreType.DMA,
        ],
    )
    def gather_kernel(table_hbm, idx_hbm, out_hbm, idx_v, rows_v, sem):
        wid = lax.axis_index("s") * info.num_cores + lax.axis_index("c")
        base = wid * b_per_w
        pltpu.sync_copy(idx_hbm.at[pl.ds(base, b_per_w)], idx_v)
        pltpu.async_copy(table_hbm.at[idx_v], rows_v, sem).wait()
        pltpu.sync_copy(rows_v, out_hbm.at[pl.ds(base, b_per_w)])

    return gather_kernel(emb, ids)


def _gains_body(ids_ref, out_ref):
    ids = ids_ref[...]
    T = ids.shape[0]
    amp = (ids % 97).astype(jnp.float32) / 97.0
    pitch = (ids % 31).astype(jnp.float32) / 31.0
    boundary = (ids % 50 == 0).astype(jnp.float32)
    g = 1.0 + 0.3 * amp + 0.2 * pitch + 0.5 * boundary
    iota = lax.broadcasted_iota(jnp.int32, g.shape, 0)
    m1 = jnp.max(g)
    i1 = jnp.min(jnp.where(g == m1, iota, T))
    g_wo = jnp.where(iota == i1, jnp.float32(-jnp.inf), g)
    m2 = jnp.max(g_wo)
    i2 = jnp.min(jnp.where(g_wo == m2, iota, T))
    out_ref[...] = g + jnp.where((iota == i1) | (iota == i2), 0.5, 0.0)


def _gains_call(T, **kw):
    return pl.pallas_call(
        _gains_body,
        out_shape=jax.ShapeDtypeStruct((T, 1), jnp.float32),
        **kw,
    )


def _encdec_body(x_ref, we_ref, be_ref, wd_ref, bd_ref, g_ref, out_ref,
                 i16_ref, c1_ref, i2_ref, u1_ref, u2_ref):
    @pl.when(pl.program_id(0) == 0)
    def _init():
        u1_ref[...] = jnp.zeros_like(u1_ref)
        u2_ref[...] = jnp.zeros_like(u2_ref)

    g = g_ref[...]
    mod1 = 1.0 + 0.3 * (g - 1.0)
    mod2 = 1.0 + 0.2 * (g - 1.0)

    I1 = jnp.dot(x_ref[...], we_ref[...],
                 preferred_element_type=jnp.float32) + be_ref[...]
    i16_ref[...] = (I1 * mod1) * float(L)

    def enc_step(i, u):
        w = DECAY * u + i16_ref[pl.ds(i, 1), :]
        c = jnp.minimum(jnp.maximum(jnp.round(w), 0.0), float(L))
        c1_ref[pl.ds(i, 1), :] = c
        return w - c

    u1_ref[...] = lax.fori_loop(0, c1_ref.shape[0], enc_step, u1_ref[...])

    d16 = jnp.dot(c1_ref[...], wd_ref[...], preferred_element_type=jnp.float32)
    i2_ref[...] = ((d16 * (1.0 / L) + bd_ref[...]) * mod2) * float(L)

    def dec_step(i, u):
        w = DECAY * u + i2_ref[pl.ds(i, 1), :]
        c = jnp.minimum(jnp.maximum(jnp.round(w), 0.0), float(L))
        out_ref[pl.ds(i, 1), :] = c * (1.0 / L)
        return w - c

    u2_ref[...] = lax.fori_loop(0, i2_ref.shape[0], dec_step, u2_ref[...])


def _encdec_call(T, E, H, **kw):
    return pl.pallas_call(
        _encdec_body,
        grid=(T // TCHUNK,),
        in_specs=[
            pl.BlockSpec((TCHUNK, E), lambda i: (i, 0)),
            pl.BlockSpec((E, H), lambda i: (0, 0)),
            pl.BlockSpec((1, H), lambda i: (0, 0)),
            pl.BlockSpec((H, E), lambda i: (0, 0)),
            pl.BlockSpec((1, E), lambda i: (0, 0)),
            pl.BlockSpec((TCHUNK, 1), lambda i: (i, 0)),
        ],
        out_specs=pl.BlockSpec((TCHUNK, E), lambda i: (i, 0)),
        out_shape=jax.ShapeDtypeStruct((T, E), jnp.float32),
        scratch_shapes=[
            pltpu.VMEM((TCHUNK, H), jnp.float32),
            pltpu.VMEM((TCHUNK, H), jnp.float32),
            pltpu.VMEM((TCHUNK, E), jnp.float32),
            pltpu.VMEM((1, H), jnp.float32),
            pltpu.VMEM((1, E), jnp.float32),
        ],
        **kw,
    )


def _logits_body(a_ref, w_ref, b_ref, out_ref):
    out_ref[...] = jnp.dot(a_ref[...], w_ref[...],
                           preferred_element_type=jnp.float32) + b_ref[...]


def _logits_call(T, E, V, NV, **kw):
    return pl.pallas_call(
        _logits_body,
        grid=(V // NV,),
        in_specs=[
            pl.BlockSpec((T, E), lambda n: (0, 0)),
            pl.BlockSpec((E, NV), lambda n: (0, n)),
            pl.BlockSpec((1, NV), lambda n: (0, n)),
        ],
        out_specs=pl.BlockSpec((T, NV), lambda n: (0, n)),
        out_shape=jax.ShapeDtypeStruct((T, V), jnp.float32),
        **kw,
    )


def kernel(input_ids, emb, W_enc, b_enc, W_dec, b_dec, W_out, b_out):
    B, T = input_ids.shape
    V, E = emb.shape
    H = W_enc.shape[1]
    assert B == 1

    ids = input_ids.reshape(T).astype(jnp.int32)
    x = _sc_gather(emb, ids)
    gains = _gains_call(T)(input_ids.reshape(T, 1).astype(jnp.int32))
    decoded = _encdec_call(T, E, H)(
        x, W_enc, b_enc.reshape(1, H), W_dec, b_dec.reshape(1, E), gains)
    logits = _logits_call(T, E, V, 512)(
        decoded.astype(jnp.bfloat16), W_out.astype(jnp.bfloat16),
        b_out.reshape(1, V))
    return logits.reshape(B, T, V)
